# baseline (speedup 1.0000x reference)
import functools

import jax
import jax.numpy as jnp
from jax import lax
from jax.experimental import pallas as pl
from jax.experimental.pallas import tpu as pltpu
from jax.experimental.pallas import tpu_sc as plsc

N = 50000
E = 800000
H = 64
L = 4

NC = 2
NS = 16
NW = NC * NS

NP = 50176
NHALF = NP // 2

E2 = E // 2
CG = 128
NCH = E2 // CG
GFULL = NCH // NW
GEXTRA = NCH - GFULL * NW
CS = 128
SFULL = NCH // NS
SEXTRA = NCH - SFULL * NS

SROWS = NHALF + NS
OPS = NHALF // NS
OCH_FULL = OPS // CS
OREM = OPS - OCH_FULL * CS


def _ln(x, g, b):
    mu = jnp.mean(x, axis=-1, keepdims=True)
    xc = x - mu
    var = jnp.mean(xc * xc, axis=-1, keepdims=True)
    return xc / jnp.sqrt(var + 1e-5) * g + b


def _gelu(x):
    return 0.5 * x * (1.0 + lax.erf(x * 0.7071067811865476))


def _gather_body(a_hbm, b_hbm, src_hbm, dst_hbm, out_hbm,
                 idxa0, idxb0, idxa1, idxb1,
                 bufa0, bufb0, bufa1, bufb1,
                 sa0, sb0, sa1, sb1):
    c = lax.axis_index("c")
    s = lax.axis_index("s")
    wid = s * NC + c
    trip = jnp.where(wid < GEXTRA, GFULL + 1, GFULL)

    idxa = (idxa0, idxa1)
    idxb = (idxb0, idxb1)
    bufa = (bufa0, bufa1)
    bufb = (bufb0, bufb1)
    sa = (sa0, sa1)
    sb = (sb0, sb1)

    def fire(k, slot):
        off = (wid + k * NW) * CG
        pltpu.sync_copy(src_hbm.at[pl.ds(off, CG)], idxa[slot])
        pltpu.sync_copy(dst_hbm.at[pl.ds(off, CG)], idxb[slot])
        pltpu.async_copy(a_hbm.at[idxa[slot]], bufa[slot], sa[slot])
        pltpu.async_copy(b_hbm.at[idxb[slot]], bufb[slot], sb[slot])

    def drain_process(k, slot):
        pltpu.make_async_copy(a_hbm.at[idxa[slot]], bufa[slot], sa[slot]).wait()
        pltpu.make_async_copy(b_hbm.at[idxb[slot]], bufb[slot], sb[slot]).wait()
        A = bufa[slot]
        B = bufb[slot]

        def addrow(r, _):
            for j in range(H // 32):
                A[r, pl.ds(j * 32, 32)] = (
                    A[r, pl.ds(j * 32, 32)] + B[r, pl.ds(j * 32, 32)])
            return 0

        lax.fori_loop(0, CG, addrow, 0)
        pltpu.sync_copy(A, out_hbm.at[pl.ds((wid + k * NW) * CG, CG)])

    fire(0, 0)

    def step(k, _):
        @pl.when(k < trip)
        def _():
            @pl.when(k + 1 < trip)
            def _():
                @pl.when((k + 1) % 2 == 0)
                def _():
                    fire(k + 1, 0)

                @pl.when((k + 1) % 2 == 1)
                def _():
                    fire(k + 1, 1)

            @pl.when(k % 2 == 0)
            def _():
                drain_process(k, 0)

            @pl.when(k % 2 == 1)
            def _():
                drain_process(k, 1)

        return 0

    lax.fori_loop(0, GFULL + 1, step, 0)


def _make_gather():
    mesh = plsc.VectorSubcoreMesh(
        core_axis_name="c", subcore_axis_name="s",
        num_cores=NC, num_subcores=NS)
    return pl.kernel(
        _gather_body,
        out_type=jax.ShapeDtypeStruct((E2, H), jnp.bfloat16),
        mesh=mesh,
        scratch_types=[
            pltpu.VMEM((CG,), jnp.int32), pltpu.VMEM((CG,), jnp.int32),
            pltpu.VMEM((CG,), jnp.int32), pltpu.VMEM((CG,), jnp.int32),
            pltpu.VMEM((CG, H), jnp.bfloat16), pltpu.VMEM((CG, H), jnp.bfloat16),
            pltpu.VMEM((CG, H), jnp.bfloat16), pltpu.VMEM((CG, H), jnp.bfloat16),
            pltpu.SemaphoreType.DMA, pltpu.SemaphoreType.DMA,
            pltpu.SemaphoreType.DMA, pltpu.SemaphoreType.DMA,
        ],
        compiler_params=pltpu.CompilerParams(use_tc_tiling_on_sc=False),
        name="gnn_edge_gather_add",
    )


def _scatter_body(m0_hbm, m1_hbm, dst0_hbm, dst1_hbm, out_hbm,
                  spmem, zbuf,
                  idxd0, idxd1, idxl0, idxl1, bufm0, bufm1,
                  sm0, sm1):
    c = lax.axis_index("c")
    s = lax.axis_index("s")
    lo = c * NHALF
    trash = NHALF + s
    trip = jnp.where(s < SEXTRA, SFULL + 1, SFULL)

    def zrow(r, _):
        for j in range(H // 16):
            zbuf[r, pl.ds(j * 16, 16)] = jnp.zeros((16,), jnp.float32)
        return 0

    lax.fori_loop(0, CS, zrow, 0)

    zb = s * OPS

    def zchunk(i, _):
        pltpu.sync_copy(zbuf, spmem.at[pl.ds(zb + i * CS, CS)])
        return 0

    lax.fori_loop(0, OCH_FULL, zchunk, 0)
    pltpu.sync_copy(zbuf.at[pl.ds(0, OREM)],
                    spmem.at[pl.ds(zb + OCH_FULL * CS, OREM)])
    plsc.subcore_barrier()

    idxd = (idxd0, idxd1)
    idxl = (idxl0, idxl1)
    bufm = (bufm0, bufm1)
    sm = (sm0, sm1)

    def run_half(m_hbm, dst_hbm):
        def fire(k, slot):
            off = (s + k * NS) * CS
            pltpu.sync_copy(dst_hbm.at[pl.ds(off, CS)], idxd[slot])
            pltpu.async_copy(m_hbm.at[pl.ds(off, CS)], bufm[slot], sm[slot])

        def drain_process(k, slot):
            def lslice(j, _):
                v = idxd[slot][pl.ds(j * 16, 16)]
                li = v - lo
                oob = (li < 0) | (li >= NHALF)
                idxl[slot][pl.ds(j * 16, 16)] = jnp.where(oob, trash, li)
                return 0

            lax.fori_loop(0, CS // 16, lslice, 0)
            pltpu.make_async_copy(
                m_hbm.at[pl.ds((s + k * NS) * CS, CS)],
                bufm[slot], sm[slot]).wait()
            pltpu.sync_copy(bufm[slot], spmem.at[idxl[slot]], add=True)

        fire(0, 0)

        def step(k, _):
            @pl.when(k < trip)
            def _():
                @pl.when(k + 1 < trip)
                def _():
                    @pl.when((k + 1) % 2 == 0)
                    def _():
                        fire(k + 1, 0)

                    @pl.when((k + 1) % 2 == 1)
                    def _():
                        fire(k + 1, 1)

                @pl.when(k % 2 == 0)
                def _():
                    drain_process(k, 0)

                @pl.when(k % 2 == 1)
                def _():
                    drain_process(k, 1)

            return 0

        lax.fori_loop(0, SFULL + 1, step, 0)

    run_half(m0_hbm, dst0_hbm)
    run_half(m1_hbm, dst1_hbm)

    plsc.subcore_barrier()

    ob = s * OPS

    def ochunk(i, _):
        pltpu.sync_copy(spmem.at[pl.ds(ob + i * CS, CS)],
                        out_hbm.at[pl.ds(lo + ob + i * CS, CS)])
        return 0

    lax.fori_loop(0, OCH_FULL, ochunk, 0)
    pltpu.sync_copy(spmem.at[pl.ds(ob + OCH_FULL * CS, OREM)],
                    out_hbm.at[pl.ds(lo + ob + OCH_FULL * CS, OREM)])


def _make_scatter():
    mesh = plsc.VectorSubcoreMesh(
        core_axis_name="c", subcore_axis_name="s",
        num_cores=NC, num_subcores=NS)
    return pl.kernel(
        _scatter_body,
        out_type=jax.ShapeDtypeStruct((NP, H), jnp.float32),
        mesh=mesh,
        scratch_types=[
            pltpu.VMEM_SHARED((SROWS, H), jnp.float32),
            pltpu.VMEM((CS, H), jnp.float32),
            pltpu.VMEM((CS,), jnp.int32), pltpu.VMEM((CS,), jnp.int32),
            pltpu.VMEM((CS,), jnp.int32), pltpu.VMEM((CS,), jnp.int32),
            pltpu.VMEM((CS, H), jnp.float32), pltpu.VMEM((CS, H), jnp.float32),
            pltpu.SemaphoreType.DMA, pltpu.SemaphoreType.DMA,
        ],
        compiler_params=pltpu.CompilerParams(use_tc_tiling_on_sc=False),
        name="gnn_scatter_add",
    )


BN = 1568
BE = 2000
BP = 3136


def _encode_body(nf, new, neb, neg, nebb, wt, wb, h_o, a_o, b_o):
    y = jnp.dot(nf[...], new[...], preferred_element_type=jnp.float32)
    h = _gelu(_ln(y + neb[...], neg[...], nebb[...]))
    h_o[...] = h
    a_o[...] = jnp.dot(
        h, wt[...], preferred_element_type=jnp.float32).astype(jnp.bfloat16)
    b_o[...] = jnp.dot(
        h, wb[...], preferred_element_type=jnp.float32).astype(jnp.bfloat16)


def _encode_call(nf, new, neb, neg, nebb, wt, wb):
    grid = NP // BN
    vec = pl.BlockSpec((1, H), lambda i: (0, 0))
    return pl.pallas_call(
        _encode_body,
        grid=(grid,),
        in_specs=[
            pl.BlockSpec((BN, 8), lambda i: (i, 0)),
            pl.BlockSpec((8, H), lambda i: (0, 0)),
            vec, vec, vec,
            pl.BlockSpec((H, H), lambda i: (0, 0)),
            pl.BlockSpec((H, H), lambda i: (0, 0)),
        ],
        out_specs=[
            pl.BlockSpec((BN, H), lambda i: (i, 0)),
            pl.BlockSpec((BN, H), lambda i: (i, 0)),
            pl.BlockSpec((BN, H), lambda i: (i, 0)),
        ],
        out_shape=[jax.ShapeDtypeStruct((NP, H), jnp.float32),
                   jax.ShapeDtypeStruct((NP, H), jnp.bfloat16),
                   jax.ShapeDtypeStruct((NP, H), jnp.bfloat16)],
    )(nf, new, neb, neg, nebb, wt, wb)


def _lngelu_body(s_ref, mb, mg, mbb, m_o):
    y = s_ref[...].astype(jnp.float32) + mb[...]
    yl = y[:, :H]
    yr = y[:, H:]
    gl = _gelu(_ln(yl, mg[...][:, :H], mbb[...][:, :H]))
    gr = _gelu(_ln(yr, mg[...][:, H:], mbb[...][:, H:]))
    m_o[...] = jnp.concatenate([gl, gr], axis=1)


def _lngelu_call(s, mb, mg, mbb):
    s2 = s.reshape(E2 // 2, 2 * H)
    mb2 = jnp.concatenate([mb, mb], axis=1)
    mg2 = jnp.concatenate([mg, mg], axis=1)
    mbb2 = jnp.concatenate([mbb, mbb], axis=1)
    grid = (E2 // 2) // BE
    vec = pl.BlockSpec((1, 2 * H), lambda i: (0, 0))
    m2 = pl.pallas_call(
        _lngelu_body,
        grid=(grid,),
        in_specs=[pl.BlockSpec((BE, 2 * H), lambda i: (i, 0)), vec, vec, vec],
        out_specs=pl.BlockSpec((BE, 2 * H), lambda i: (i, 0)),
        out_shape=jax.ShapeDtypeStruct((E2 // 2, 2 * H), jnp.float32),
    )(s2, mb2, mg2, mbb2)
    return m2.reshape(E2, H)


def _update_body(h_ref, hn_ref, ng, nb, wt, wb, h_o, a_o, b_o):
    h = _ln(h_ref[...] + hn_ref[...], ng[...], nb[...])
    h_o[...] = h
    a_o[...] = jnp.dot(
        h, wt[...], preferred_element_type=jnp.float32).astype(jnp.bfloat16)
    b_o[...] = jnp.dot(
        h, wb[...], preferred_element_type=jnp.float32).astype(jnp.bfloat16)


def _update_call(h, hn, ng, nb, wt, wb):
    grid = NP // BN
    vec = pl.BlockSpec((1, H), lambda i: (0, 0))
    return pl.pallas_call(
        _update_body,
        grid=(grid,),
        in_specs=[
            pl.BlockSpec((BN, H), lambda i: (i, 0)),
            pl.BlockSpec((BN, H), lambda i: (i, 0)),
            vec, vec,
            pl.BlockSpec((H, H), lambda i: (0, 0)),
            pl.BlockSpec((H, H), lambda i: (0, 0)),
        ],
        out_specs=[
            pl.BlockSpec((BN, H), lambda i: (i, 0)),
            pl.BlockSpec((BN, H), lambda i: (i, 0)),
            pl.BlockSpec((BN, H), lambda i: (i, 0)),
        ],
        out_shape=[jax.ShapeDtypeStruct((NP, H), jnp.float32),
                   jax.ShapeDtypeStruct((NP, H), jnp.bfloat16),
                   jax.ShapeDtypeStruct((NP, H), jnp.bfloat16)],
    )(h, hn, ng, nb, wt, wb)


def _update_last_body(h_ref, hn_ref, ng, nb, h_o):
    h_o[...] = _ln(h_ref[...] + hn_ref[...], ng[...], nb[...])


def _update_last_call(h, hn, ng, nb):
    grid = NP // BN
    vec = pl.BlockSpec((1, H), lambda i: (0, 0))
    return pl.pallas_call(
        _update_last_body,
        grid=(grid,),
        in_specs=[
            pl.BlockSpec((BN, H), lambda i: (i, 0)),
            pl.BlockSpec((BN, H), lambda i: (i, 0)),
            vec, vec,
        ],
        out_specs=pl.BlockSpec((BN, H), lambda i: (i, 0)),
        out_shape=jax.ShapeDtypeStruct((NP, H), jnp.float32),
    )(h, hn, ng, nb)


def _pool_body(h_ref, nt_ref, gf, gew, geb, geg, gebb,
               pw1, pb1, pg1, pbb1, pw2, pb2, hw1, hb1, hw2, hb2,
               out_ref, acc_ref):
    i = pl.program_id(0)

    @pl.when(i == 0)
    def _():
        acc_ref[...] = jnp.zeros_like(acc_ref)

    h = h_ref[...]
    nt = nt_ref[...]
    qm = (nt == 0).astype(jnp.float32)
    cm = (nt == 1).astype(jnp.float32)
    acc_ref[0:1, 0:H] += jnp.sum(h * qm, axis=0, keepdims=True)
    acc_ref[1:2, 0:H] += jnp.sum(h * cm, axis=0, keepdims=True)
    acc_ref[2:3, 0:1] += jnp.sum(qm, keepdims=True)
    acc_ref[3:4, 0:1] += jnp.sum(cm, keepdims=True)

    @pl.when(i == NP // BP - 1)
    def _():
        qc = jnp.maximum(acc_ref[2, 0], 1.0)
        cc = jnp.maximum(acc_ref[3, 0], 1.0)
        q_pool = acc_ref[0:1, 0:H] / qc
        c_pool = acc_ref[1:2, 0:H] / cc
        gy = jnp.dot(gf[...], gew[...], preferred_element_type=jnp.float32)
        g = _gelu(_ln(gy + geb[...], geg[...], gebb[...]))
        combined = jnp.concatenate([q_pool, c_pool, g], axis=1)
        f1y = jnp.dot(combined, pw1[...], preferred_element_type=jnp.float32)
        f1 = _gelu(_ln(f1y + pb1[...], pg1[...], pbb1[...]))
        feats = _gelu(
            jnp.dot(f1, pw2[...], preferred_element_type=jnp.float32)
            + pb2[...])
        vals = []
        for hd in range(4):
            t = _gelu(
                jnp.dot(feats, hw1[hd], preferred_element_type=jnp.float32)
                + hb1[hd])
            vals.append(
                jnp.dot(t, hw2[hd], preferred_element_type=jnp.float32)
                + hb2[hd])
        acc = jax.nn.sigmoid(vals[0])
        en = jax.nn.softplus(vals[1])
        tr = jax.nn.sigmoid(vals[2])
        dp = jax.nn.softplus(vals[3])
        row = jnp.concatenate(
            [acc, en, tr, dp, jnp.zeros((1, 124), jnp.float32)], axis=1)
        out_ref[...] = row


def _pool_call(h, nt, gf, gew, geb, geg, gebb,
               pw1, pb1, pg1, pbb1, pw2, pb2, hw1, hb1, hw2, hb2):
    grid = NP // BP
    vec = pl.BlockSpec((1, H), lambda i: (0, 0))
    full = lambda shape: pl.BlockSpec(shape, lambda i: tuple(0 for _ in shape))
    return pl.pallas_call(
        _pool_body,
        grid=(grid,),
        in_specs=[
            pl.BlockSpec((BP, H), lambda i: (i, 0)),
            pl.BlockSpec((BP, 1), lambda i: (i, 0)),
            full((1, 8)), full((8, H)), vec, vec, vec,
            full((3 * H, 2 * H)), full((1, 2 * H)), full((1, 2 * H)),
            full((1, 2 * H)),
            full((2 * H, H)), full((1, H)),
            full((4, H, H // 2)), full((4, 1, H // 2)),
            full((4, H // 2, 1)), full((4, 1, 1)),
        ],
        out_specs=pl.BlockSpec((1, 128), lambda i: (0, 0)),
        out_shape=jax.ShapeDtypeStruct((1, 128), jnp.float32),
        scratch_shapes=[pltpu.VMEM((8, 128), jnp.float32)],
    )(h, nt, gf, gew, geb, geg, gebb,
      pw1, pb1, pg1, pbb1, pw2, pb2, hw1, hb1, hw2, hb2)


def kernel(node_features, edge_attr, global_features, edge_index, node_types,
           ne_w, ne_b, ne_g, ne_bb, ee_w, ee_b, ee_g, ee_bb,
           ge_w, ge_b, ge_g, ge_bb, msg_w, msg_b, msg_g, msg_bb,
           norm_g, norm_b, pw1, pb1, pg1, pbb1, pw2, pb2,
           hw1, hb1, hw2, hb2):
    del edge_attr, ee_w, ee_b, ee_g, ee_bb

    src = edge_index[0]
    dst = edge_index[1]
    nf_pad = jnp.zeros((NP, 8), jnp.float32).at[:N].set(node_features)
    nt_pad = jnp.full((NP, 1), 2, jnp.int32).at[:N, 0].set(node_types)

    r1 = lambda v: v.reshape(1, -1)
    wts = [msg_w[i, :H, :] for i in range(L)]
    wbs = [msg_w[i, H:, :] for i in range(L)]

    gather = _make_gather()
    scatter = _make_scatter()

    src0, src1 = src[:E2], src[E2:]
    dst0, dst1 = dst[:E2], dst[E2:]

    h, A, B = _encode_call(nf_pad, ne_w, r1(ne_b), r1(ne_g), r1(ne_bb),
                           wts[0], wbs[0])
    for i in range(L):
        S0 = gather(A, B, src0, dst0)
        S1 = gather(A, B, src1, dst1)
        M0 = _lngelu_call(S0, r1(msg_b[i]), r1(msg_g[i]), r1(msg_bb[i]))
        M1 = _lngelu_call(S1, r1(msg_b[i]), r1(msg_g[i]), r1(msg_bb[i]))
        hn = scatter(M0, M1, dst0, dst1)
        if i < L - 1:
            h, A, B = _update_call(h, hn, r1(norm_g[i]), r1(norm_b[i]),
                                   wts[i + 1], wbs[i + 1])
        else:
            h = _update_last_call(h, hn, r1(norm_g[i]), r1(norm_b[i]))

    pooled = _pool_call(h, nt_pad, global_features.reshape(1, 8),
                        ge_w, r1(ge_b), r1(ge_g), r1(ge_bb),
                        pw1, r1(pb1), r1(pg1), r1(pbb1), pw2, r1(pb2),
                        hw1, hb1.reshape(4, 1, H // 2), hw2,
                        hb2.reshape(4, 1, 1))
    return pooled[0, :4]


# speedup vs baseline: 1.3815x; 1.3815x over previous
import functools

import jax
import jax.numpy as jnp
from jax import lax
from jax.experimental import pallas as pl
from jax.experimental.pallas import tpu as pltpu
from jax.experimental.pallas import tpu_sc as plsc

N = 50000
E = 800000
H = 64
L = 4

NC = 2
NS = 16
NW = NC * NS

NP = 50176
NHALF = NP // 2

E2 = E // 2
CG = 128
NCH = E2 // CG
GFULL = NCH // NW
GEXTRA = NCH - GFULL * NW
CS = 128
SFULL = NCH // NS
SEXTRA = NCH - SFULL * NS

SROWS = NHALF + NS
OPS = NHALF // NS
OCH_FULL = OPS // CS
OREM = OPS - OCH_FULL * CS


def _ln(x, g, b):
    mu = jnp.mean(x, axis=-1, keepdims=True)
    xc = x - mu
    var = jnp.mean(xc * xc, axis=-1, keepdims=True)
    return xc / jnp.sqrt(var + 1e-5) * g + b


def _gelu(x):
    return 0.5 * x * (1.0 + lax.erf(x * 0.7071067811865476))


def _gather_body(a_hbm, b_hbm, src_hbm, dst_hbm, out_hbm,
                 idxa0, idxb0, idxa1, idxb1,
                 bufa0, bufb0, bufa1, bufb1,
                 sa0, sb0, sa1, sb1):
    c = lax.axis_index("c")
    s = lax.axis_index("s")
    wid = s * NC + c
    trip = jnp.where(wid < GEXTRA, GFULL + 1, GFULL)

    idxa = (idxa0, idxa1)
    idxb = (idxb0, idxb1)
    bufa = (bufa0, bufa1)
    bufb = (bufb0, bufb1)
    sa = (sa0, sa1)
    sb = (sb0, sb1)

    def fire(k, slot):
        off = (wid + k * NW) * CG
        pltpu.sync_copy(src_hbm.at[pl.ds(off, CG)], idxa[slot])
        pltpu.sync_copy(dst_hbm.at[pl.ds(off, CG)], idxb[slot])
        pltpu.async_copy(a_hbm.at[idxa[slot]], bufa[slot], sa[slot])
        pltpu.async_copy(b_hbm.at[idxb[slot]], bufb[slot], sb[slot])

    def drain_process(k, slot):
        pltpu.make_async_copy(a_hbm.at[idxa[slot]], bufa[slot], sa[slot]).wait()
        pltpu.make_async_copy(b_hbm.at[idxb[slot]], bufb[slot], sb[slot]).wait()
        A = bufa[slot]
        B = bufb[slot]

        def addrow(r, _):
            for j in range(H // 16):
                A[r, pl.ds(j * 16, 16)] = (
                    A[r, pl.ds(j * 16, 16)] + B[r, pl.ds(j * 16, 16)])
            return 0

        lax.fori_loop(0, CG, addrow, 0)
        pltpu.sync_copy(A, out_hbm.at[pl.ds((wid + k * NW) * CG, CG)])

    fire(0, 0)

    def step(k, _):
        @pl.when(k < trip)
        def _():
            @pl.when(k + 1 < trip)
            def _():
                @pl.when((k + 1) % 2 == 0)
                def _():
                    fire(k + 1, 0)

                @pl.when((k + 1) % 2 == 1)
                def _():
                    fire(k + 1, 1)

            @pl.when(k % 2 == 0)
            def _():
                drain_process(k, 0)

            @pl.when(k % 2 == 1)
            def _():
                drain_process(k, 1)

        return 0

    lax.fori_loop(0, GFULL + 1, step, 0)


def _make_gather():
    mesh = plsc.VectorSubcoreMesh(
        core_axis_name="c", subcore_axis_name="s",
        num_cores=NC, num_subcores=NS)
    return pl.kernel(
        _gather_body,
        out_type=jax.ShapeDtypeStruct((E2, H), jnp.float32),
        mesh=mesh,
        scratch_types=[
            pltpu.VMEM((CG,), jnp.int32), pltpu.VMEM((CG,), jnp.int32),
            pltpu.VMEM((CG,), jnp.int32), pltpu.VMEM((CG,), jnp.int32),
            pltpu.VMEM((CG, H), jnp.float32), pltpu.VMEM((CG, H), jnp.float32),
            pltpu.VMEM((CG, H), jnp.float32), pltpu.VMEM((CG, H), jnp.float32),
            pltpu.SemaphoreType.DMA, pltpu.SemaphoreType.DMA,
            pltpu.SemaphoreType.DMA, pltpu.SemaphoreType.DMA,
        ],
        compiler_params=pltpu.CompilerParams(use_tc_tiling_on_sc=False),
        name="gnn_edge_gather_add",
    )


def _scatter_body(m0_hbm, m1_hbm, dst0_hbm, dst1_hbm, out_hbm,
                  spmem, zbuf,
                  idxd0, idxd1, idxl0, idxl1, bufm0, bufm1,
                  sm0, sm1):
    c = lax.axis_index("c")
    s = lax.axis_index("s")
    lo = c * NHALF
    trash = NHALF + s
    trip = jnp.where(s < SEXTRA, SFULL + 1, SFULL)

    def zrow(r, _):
        for j in range(H // 16):
            zbuf[r, pl.ds(j * 16, 16)] = jnp.zeros((16,), jnp.float32)
        return 0

    lax.fori_loop(0, CS, zrow, 0)

    zb = s * OPS

    def zchunk(i, _):
        pltpu.sync_copy(zbuf, spmem.at[pl.ds(zb + i * CS, CS)])
        return 0

    lax.fori_loop(0, OCH_FULL, zchunk, 0)
    pltpu.sync_copy(zbuf.at[pl.ds(0, OREM)],
                    spmem.at[pl.ds(zb + OCH_FULL * CS, OREM)])
    plsc.subcore_barrier()

    idxd = (idxd0, idxd1)
    idxl = (idxl0, idxl1)
    bufm = (bufm0, bufm1)
    sm = (sm0, sm1)

    def run_half(m_hbm, dst_hbm):
        def fire(k, slot):
            off = (s + k * NS) * CS
            pltpu.sync_copy(dst_hbm.at[pl.ds(off, CS)], idxd[slot])
            pltpu.async_copy(m_hbm.at[pl.ds(off, CS)], bufm[slot], sm[slot])

        def drain_process(k, slot):
            def lslice(j, _):
                v = idxd[slot][pl.ds(j * 16, 16)]
                li = v - lo
                oob = (li < 0) | (li >= NHALF)
                idxl[slot][pl.ds(j * 16, 16)] = jnp.where(oob, trash, li)
                return 0

            lax.fori_loop(0, CS // 16, lslice, 0)
            pltpu.make_async_copy(
                m_hbm.at[pl.ds((s + k * NS) * CS, CS)],
                bufm[slot], sm[slot]).wait()
            pltpu.sync_copy(bufm[slot], spmem.at[idxl[slot]], add=True)

        fire(0, 0)

        def step(k, _):
            @pl.when(k < trip)
            def _():
                @pl.when(k + 1 < trip)
                def _():
                    @pl.when((k + 1) % 2 == 0)
                    def _():
                        fire(k + 1, 0)

                    @pl.when((k + 1) % 2 == 1)
                    def _():
                        fire(k + 1, 1)

                @pl.when(k % 2 == 0)
                def _():
                    drain_process(k, 0)

                @pl.when(k % 2 == 1)
                def _():
                    drain_process(k, 1)

            return 0

        lax.fori_loop(0, SFULL + 1, step, 0)

    run_half(m0_hbm, dst0_hbm)
    run_half(m1_hbm, dst1_hbm)

    plsc.subcore_barrier()

    ob = s * OPS

    def ochunk(i, _):
        pltpu.sync_copy(spmem.at[pl.ds(ob + i * CS, CS)],
                        out_hbm.at[pl.ds(lo + ob + i * CS, CS)])
        return 0

    lax.fori_loop(0, OCH_FULL, ochunk, 0)
    pltpu.sync_copy(spmem.at[pl.ds(ob + OCH_FULL * CS, OREM)],
                    out_hbm.at[pl.ds(lo + ob + OCH_FULL * CS, OREM)])


def _make_scatter():
    mesh = plsc.VectorSubcoreMesh(
        core_axis_name="c", subcore_axis_name="s",
        num_cores=NC, num_subcores=NS)
    return pl.kernel(
        _scatter_body,
        out_type=jax.ShapeDtypeStruct((NP, H), jnp.float32),
        mesh=mesh,
        scratch_types=[
            pltpu.VMEM_SHARED((SROWS, H), jnp.float32),
            pltpu.VMEM((CS, H), jnp.float32),
            pltpu.VMEM((CS,), jnp.int32), pltpu.VMEM((CS,), jnp.int32),
            pltpu.VMEM((CS,), jnp.int32), pltpu.VMEM((CS,), jnp.int32),
            pltpu.VMEM((CS, H), jnp.float32), pltpu.VMEM((CS, H), jnp.float32),
            pltpu.SemaphoreType.DMA, pltpu.SemaphoreType.DMA,
        ],
        compiler_params=pltpu.CompilerParams(use_tc_tiling_on_sc=False),
        name="gnn_scatter_add",
    )


BN = 1568
BE = 2000
BP = 3136


def _encode_body(nf, new, neb, neg, nebb, wt, wb, h_o, a_o, b_o):
    y = jnp.dot(nf[...], new[...], preferred_element_type=jnp.float32)
    h = _gelu(_ln(y + neb[...], neg[...], nebb[...]))
    h_o[...] = h
    a_o[...] = jnp.dot(h, wt[...], preferred_element_type=jnp.float32)
    b_o[...] = jnp.dot(h, wb[...], preferred_element_type=jnp.float32)


def _encode_call(nf, new, neb, neg, nebb, wt, wb):
    grid = NP // BN
    vec = pl.BlockSpec((1, H), lambda i: (0, 0))
    return pl.pallas_call(
        _encode_body,
        grid=(grid,),
        in_specs=[
            pl.BlockSpec((BN, 8), lambda i: (i, 0)),
            pl.BlockSpec((8, H), lambda i: (0, 0)),
            vec, vec, vec,
            pl.BlockSpec((H, H), lambda i: (0, 0)),
            pl.BlockSpec((H, H), lambda i: (0, 0)),
        ],
        out_specs=[
            pl.BlockSpec((BN, H), lambda i: (i, 0)),
            pl.BlockSpec((BN, H), lambda i: (i, 0)),
            pl.BlockSpec((BN, H), lambda i: (i, 0)),
        ],
        out_shape=[jax.ShapeDtypeStruct((NP, H), jnp.float32)] * 3,
    )(nf, new, neb, neg, nebb, wt, wb)


def _lngelu_body(s_ref, mb, mg, mbb, m_o):
    y = s_ref[...] + mb[...]
    yl = y[:, :H]
    yr = y[:, H:]
    gl = _gelu(_ln(yl, mg[...][:, :H], mbb[...][:, :H]))
    gr = _gelu(_ln(yr, mg[...][:, H:], mbb[...][:, H:]))
    m_o[...] = jnp.concatenate([gl, gr], axis=1)


def _lngelu_call(s, mb, mg, mbb):
    s2 = s.reshape(E2 // 2, 2 * H)
    mb2 = jnp.concatenate([mb, mb], axis=1)
    mg2 = jnp.concatenate([mg, mg], axis=1)
    mbb2 = jnp.concatenate([mbb, mbb], axis=1)
    grid = (E2 // 2) // BE
    vec = pl.BlockSpec((1, 2 * H), lambda i: (0, 0))
    m2 = pl.pallas_call(
        _lngelu_body,
        grid=(grid,),
        in_specs=[pl.BlockSpec((BE, 2 * H), lambda i: (i, 0)), vec, vec, vec],
        out_specs=pl.BlockSpec((BE, 2 * H), lambda i: (i, 0)),
        out_shape=jax.ShapeDtypeStruct((E2 // 2, 2 * H), jnp.float32),
    )(s2, mb2, mg2, mbb2)
    return m2.reshape(E2, H)


def _update_body(h_ref, hn_ref, ng, nb, wt, wb, h_o, a_o, b_o):
    h = _ln(h_ref[...] + hn_ref[...], ng[...], nb[...])
    h_o[...] = h
    a_o[...] = jnp.dot(h, wt[...], preferred_element_type=jnp.float32)
    b_o[...] = jnp.dot(h, wb[...], preferred_element_type=jnp.float32)


def _update_call(h, hn, ng, nb, wt, wb):
    grid = NP // BN
    vec = pl.BlockSpec((1, H), lambda i: (0, 0))
    return pl.pallas_call(
        _update_body,
        grid=(grid,),
        in_specs=[
            pl.BlockSpec((BN, H), lambda i: (i, 0)),
            pl.BlockSpec((BN, H), lambda i: (i, 0)),
            vec, vec,
            pl.BlockSpec((H, H), lambda i: (0, 0)),
            pl.BlockSpec((H, H), lambda i: (0, 0)),
        ],
        out_specs=[
            pl.BlockSpec((BN, H), lambda i: (i, 0)),
            pl.BlockSpec((BN, H), lambda i: (i, 0)),
            pl.BlockSpec((BN, H), lambda i: (i, 0)),
        ],
        out_shape=[jax.ShapeDtypeStruct((NP, H), jnp.float32)] * 3,
    )(h, hn, ng, nb, wt, wb)


def _update_last_body(h_ref, hn_ref, ng, nb, h_o):
    h_o[...] = _ln(h_ref[...] + hn_ref[...], ng[...], nb[...])


def _update_last_call(h, hn, ng, nb):
    grid = NP // BN
    vec = pl.BlockSpec((1, H), lambda i: (0, 0))
    return pl.pallas_call(
        _update_last_body,
        grid=(grid,),
        in_specs=[
            pl.BlockSpec((BN, H), lambda i: (i, 0)),
            pl.BlockSpec((BN, H), lambda i: (i, 0)),
            vec, vec,
        ],
        out_specs=pl.BlockSpec((BN, H), lambda i: (i, 0)),
        out_shape=jax.ShapeDtypeStruct((NP, H), jnp.float32),
    )(h, hn, ng, nb)


def _pool_body(h_ref, nt_ref, gf, gew, geb, geg, gebb,
               pw1, pb1, pg1, pbb1, pw2, pb2, hw1, hb1, hw2, hb2,
               out_ref, acc_ref):
    i = pl.program_id(0)

    @pl.when(i == 0)
    def _():
        acc_ref[...] = jnp.zeros_like(acc_ref)

    h = h_ref[...]
    nt = nt_ref[...]
    qm = (nt == 0).astype(jnp.float32)
    cm = (nt == 1).astype(jnp.float32)
    acc_ref[0:1, 0:H] += jnp.sum(h * qm, axis=0, keepdims=True)
    acc_ref[1:2, 0:H] += jnp.sum(h * cm, axis=0, keepdims=True)
    acc_ref[2:3, 0:1] += jnp.sum(qm, keepdims=True)
    acc_ref[3:4, 0:1] += jnp.sum(cm, keepdims=True)

    @pl.when(i == NP // BP - 1)
    def _():
        qc = jnp.maximum(acc_ref[2, 0], 1.0)
        cc = jnp.maximum(acc_ref[3, 0], 1.0)
        q_pool = acc_ref[0:1, 0:H] / qc
        c_pool = acc_ref[1:2, 0:H] / cc
        gy = jnp.dot(gf[...], gew[...], preferred_element_type=jnp.float32)
        g = _gelu(_ln(gy + geb[...], geg[...], gebb[...]))
        combined = jnp.concatenate([q_pool, c_pool, g], axis=1)
        f1y = jnp.dot(combined, pw1[...], preferred_element_type=jnp.float32)
        f1 = _gelu(_ln(f1y + pb1[...], pg1[...], pbb1[...]))
        feats = _gelu(
            jnp.dot(f1, pw2[...], preferred_element_type=jnp.float32)
            + pb2[...])
        vals = []
        for hd in range(4):
            t = _gelu(
                jnp.dot(feats, hw1[hd], preferred_element_type=jnp.float32)
                + hb1[hd])
            vals.append(
                jnp.dot(t, hw2[hd], preferred_element_type=jnp.float32)
                + hb2[hd])
        acc = jax.nn.sigmoid(vals[0])
        en = jax.nn.softplus(vals[1])
        tr = jax.nn.sigmoid(vals[2])
        dp = jax.nn.softplus(vals[3])
        row = jnp.concatenate(
            [acc, en, tr, dp, jnp.zeros((1, 124), jnp.float32)], axis=1)
        out_ref[...] = row


def _pool_call(h, nt, gf, gew, geb, geg, gebb,
               pw1, pb1, pg1, pbb1, pw2, pb2, hw1, hb1, hw2, hb2):
    grid = NP // BP
    vec = pl.BlockSpec((1, H), lambda i: (0, 0))
    full = lambda shape: pl.BlockSpec(shape, lambda i: tuple(0 for _ in shape))
    return pl.pallas_call(
        _pool_body,
        grid=(grid,),
        in_specs=[
            pl.BlockSpec((BP, H), lambda i: (i, 0)),
            pl.BlockSpec((BP, 1), lambda i: (i, 0)),
            full((1, 8)), full((8, H)), vec, vec, vec,
            full((3 * H, 2 * H)), full((1, 2 * H)), full((1, 2 * H)),
            full((1, 2 * H)),
            full((2 * H, H)), full((1, H)),
            full((4, H, H // 2)), full((4, 1, H // 2)),
            full((4, H // 2, 1)), full((4, 1, 1)),
        ],
        out_specs=pl.BlockSpec((1, 128), lambda i: (0, 0)),
        out_shape=jax.ShapeDtypeStruct((1, 128), jnp.float32),
        scratch_shapes=[pltpu.VMEM((8, 128), jnp.float32)],
    )(h, nt, gf, gew, geb, geg, gebb,
      pw1, pb1, pg1, pbb1, pw2, pb2, hw1, hb1, hw2, hb2)


def kernel(node_features, edge_attr, global_features, edge_index, node_types,
           ne_w, ne_b, ne_g, ne_bb, ee_w, ee_b, ee_g, ee_bb,
           ge_w, ge_b, ge_g, ge_bb, msg_w, msg_b, msg_g, msg_bb,
           norm_g, norm_b, pw1, pb1, pg1, pbb1, pw2, pb2,
           hw1, hb1, hw2, hb2):
    del edge_attr, ee_w, ee_b, ee_g, ee_bb

    src = edge_index[0]
    dst = edge_index[1]
    nf_pad = jnp.zeros((NP, 8), jnp.float32).at[:N].set(node_features)
    nt_pad = jnp.full((NP, 1), 2, jnp.int32).at[:N, 0].set(node_types)

    r1 = lambda v: v.reshape(1, -1)
    wts = [msg_w[i, :H, :] for i in range(L)]
    wbs = [msg_w[i, H:, :] for i in range(L)]

    gather = _make_gather()
    scatter = _make_scatter()

    src0, src1 = src[:E2], src[E2:]
    dst0, dst1 = dst[:E2], dst[E2:]

    h, A, B = _encode_call(nf_pad, ne_w, r1(ne_b), r1(ne_g), r1(ne_bb),
                           wts[0], wbs[0])
    for i in range(L):
        S0 = gather(A, B, src0, dst0)
        S1 = gather(A, B, src1, dst1)
        M0 = _lngelu_call(S0, r1(msg_b[i]), r1(msg_g[i]), r1(msg_bb[i]))
        M1 = _lngelu_call(S1, r1(msg_b[i]), r1(msg_g[i]), r1(msg_bb[i]))
        hn = scatter(M0, M1, dst0, dst1)
        if i < L - 1:
            h, A, B = _update_call(h, hn, r1(norm_g[i]), r1(norm_b[i]),
                                   wts[i + 1], wbs[i + 1])
        else:
            h = _update_last_call(h, hn, r1(norm_g[i]), r1(norm_b[i]))

    pooled = _pool_call(h, nt_pad, global_features.reshape(1, 8),
                        ge_w, r1(ge_b), r1(ge_g), r1(ge_bb),
                        pw1, r1(pb1), r1(pg1), r1(pbb1), pw2, r1(pb2),
                        hw1, hb1.reshape(4, 1, H // 2), hw2,
                        hb2.reshape(4, 1, 1))
    return pooled[0, :4]


# speedup vs baseline: 1.3860x; 1.0032x over previous
import functools

import jax
import jax.numpy as jnp
from jax import lax
from jax.experimental import pallas as pl
from jax.experimental.pallas import tpu as pltpu
from jax.experimental.pallas import tpu_sc as plsc

N = 50000
E = 800000
H = 64
L = 4

NC = 2
NS = 16
NW = NC * NS

NP = 50176
NHALF = NP // 2

E2 = E // 2
CG = 128
NCH = E2 // CG
GFULL = NCH // NW
GEXTRA = NCH - GFULL * NW
CS = 128
SFULL = NCH // NS
SEXTRA = NCH - SFULL * NS

SROWS = NHALF + NS
OPS = NHALF // NS
OCH_FULL = OPS // CS
OREM = OPS - OCH_FULL * CS


def _ln(x, g, b):
    mu = jnp.mean(x, axis=-1, keepdims=True)
    xc = x - mu
    var = jnp.mean(xc * xc, axis=-1, keepdims=True)
    return xc / jnp.sqrt(var + 1e-5) * g + b


def _gelu(x):
    return 0.5 * x * (1.0 + lax.erf(x * 0.7071067811865476))


def _gather_body(a_hbm, b_hbm, src_hbm, dst_hbm, out_hbm,
                 idxa0, idxb0, idxa1, idxb1,
                 bufa0, bufb0, bufa1, bufb1,
                 sa0, sb0, sa1, sb1):
    c = lax.axis_index("c")
    s = lax.axis_index("s")
    wid = s * NC + c
    trip = jnp.where(wid < GEXTRA, GFULL + 1, GFULL)

    idxa = (idxa0, idxa1)
    idxb = (idxb0, idxb1)
    bufa = (bufa0, bufa1)
    bufb = (bufb0, bufb1)
    sa = (sa0, sa1)
    sb = (sb0, sb1)

    def fire(k, slot):
        off = (wid + k * NW) * CG
        pltpu.sync_copy(src_hbm.at[pl.ds(off, CG)], idxa[slot])
        pltpu.sync_copy(dst_hbm.at[pl.ds(off, CG)], idxb[slot])
        pltpu.async_copy(a_hbm.at[idxa[slot]], bufa[slot], sa[slot])
        pltpu.async_copy(b_hbm.at[idxb[slot]], bufb[slot], sb[slot])

    def drain_process(k, slot):
        pltpu.make_async_copy(a_hbm.at[idxa[slot]], bufa[slot], sa[slot]).wait()
        pltpu.make_async_copy(b_hbm.at[idxb[slot]], bufb[slot], sb[slot]).wait()
        A = bufa[slot]
        B = bufb[slot]

        def addrow(r, _):
            for j in range(H // 16):
                A[r, pl.ds(j * 16, 16)] = (
                    A[r, pl.ds(j * 16, 16)] + B[r, pl.ds(j * 16, 16)])
            return 0

        lax.fori_loop(0, CG, addrow, 0)
        pltpu.sync_copy(A, out_hbm.at[pl.ds((wid + k * NW) * CG, CG)])

    fire(0, 0)

    def step(k, _):
        @pl.when(k < trip)
        def _():
            @pl.when(k + 1 < trip)
            def _():
                @pl.when((k + 1) % 2 == 0)
                def _():
                    fire(k + 1, 0)

                @pl.when((k + 1) % 2 == 1)
                def _():
                    fire(k + 1, 1)

            @pl.when(k % 2 == 0)
            def _():
                drain_process(k, 0)

            @pl.when(k % 2 == 1)
            def _():
                drain_process(k, 1)

        return 0

    lax.fori_loop(0, GFULL + 1, step, 0)


def _make_gather():
    mesh = plsc.VectorSubcoreMesh(
        core_axis_name="c", subcore_axis_name="s",
        num_cores=NC, num_subcores=NS)
    return pl.kernel(
        _gather_body,
        out_type=jax.ShapeDtypeStruct((E2, H), jnp.float32),
        mesh=mesh,
        scratch_types=[
            pltpu.VMEM((CG,), jnp.int32), pltpu.VMEM((CG,), jnp.int32),
            pltpu.VMEM((CG,), jnp.int32), pltpu.VMEM((CG,), jnp.int32),
            pltpu.VMEM((CG, H), jnp.float32), pltpu.VMEM((CG, H), jnp.float32),
            pltpu.VMEM((CG, H), jnp.float32), pltpu.VMEM((CG, H), jnp.float32),
            pltpu.SemaphoreType.DMA, pltpu.SemaphoreType.DMA,
            pltpu.SemaphoreType.DMA, pltpu.SemaphoreType.DMA,
        ],
        compiler_params=pltpu.CompilerParams(use_tc_tiling_on_sc=False),
        name="gnn_edge_gather_add",
    )


def _scatter_body(m0_hbm, m1_hbm, dst0_hbm, dst1_hbm, out_hbm,
                  spmem, zbuf,
                  idxd0, idxd1, idxl0, idxl1, bufm0, bufm1,
                  sm0, sm1):
    c = lax.axis_index("c")
    s = lax.axis_index("s")
    lo = c * NHALF
    trash = NHALF + s
    trip = jnp.where(s < SEXTRA, SFULL + 1, SFULL)

    def zrow(r, _):
        for j in range(H // 16):
            zbuf[r, pl.ds(j * 16, 16)] = jnp.zeros((16,), jnp.float32)
        return 0

    lax.fori_loop(0, CS, zrow, 0)

    zb = s * OPS

    def zchunk(i, _):
        pltpu.sync_copy(zbuf, spmem.at[pl.ds(zb + i * CS, CS)])
        return 0

    lax.fori_loop(0, OCH_FULL, zchunk, 0)
    pltpu.sync_copy(zbuf.at[pl.ds(0, OREM)],
                    spmem.at[pl.ds(zb + OCH_FULL * CS, OREM)])
    plsc.subcore_barrier()

    idxd = (idxd0, idxd1)
    idxl = (idxl0, idxl1)
    bufm = (bufm0, bufm1)
    sm = (sm0, sm1)

    def run_half(m_hbm, dst_hbm):
        def fire(k, slot):
            off = (s + k * NS) * CS
            pltpu.sync_copy(dst_hbm.at[pl.ds(off, CS)], idxd[slot])
            pltpu.async_copy(m_hbm.at[pl.ds(off, CS)], bufm[slot], sm[slot])

        def drain_process(k, slot):
            def lslice(j, _):
                v = idxd[slot][pl.ds(j * 16, 16)]
                li = v - lo
                oob = (li < 0) | (li >= NHALF)
                idxl[slot][pl.ds(j * 16, 16)] = jnp.where(oob, trash, li)
                return 0

            lax.fori_loop(0, CS // 16, lslice, 0)
            pltpu.make_async_copy(
                m_hbm.at[pl.ds((s + k * NS) * CS, CS)],
                bufm[slot], sm[slot]).wait()
            pltpu.sync_copy(bufm[slot], spmem.at[idxl[slot]], add=True)

        fire(0, 0)

        def step(k, _):
            @pl.when(k < trip)
            def _():
                @pl.when(k + 1 < trip)
                def _():
                    @pl.when((k + 1) % 2 == 0)
                    def _():
                        fire(k + 1, 0)

                    @pl.when((k + 1) % 2 == 1)
                    def _():
                        fire(k + 1, 1)

                @pl.when(k % 2 == 0)
                def _():
                    drain_process(k, 0)

                @pl.when(k % 2 == 1)
                def _():
                    drain_process(k, 1)

            return 0

        lax.fori_loop(0, SFULL + 1, step, 0)

    run_half(m0_hbm, dst0_hbm)
    run_half(m1_hbm, dst1_hbm)

    plsc.subcore_barrier()

    ob = s * OPS

    def ochunk(i, _):
        pltpu.sync_copy(spmem.at[pl.ds(ob + i * CS, CS)],
                        out_hbm.at[pl.ds(lo + ob + i * CS, CS)])
        return 0

    lax.fori_loop(0, OCH_FULL, ochunk, 0)
    pltpu.sync_copy(spmem.at[pl.ds(ob + OCH_FULL * CS, OREM)],
                    out_hbm.at[pl.ds(lo + ob + OCH_FULL * CS, OREM)])


def _make_scatter():
    mesh = plsc.VectorSubcoreMesh(
        core_axis_name="c", subcore_axis_name="s",
        num_cores=NC, num_subcores=NS)
    return pl.kernel(
        _scatter_body,
        out_type=jax.ShapeDtypeStruct((NP, H), jnp.float32),
        mesh=mesh,
        scratch_types=[
            pltpu.VMEM_SHARED((SROWS, H), jnp.float32),
            pltpu.VMEM((CS, H), jnp.float32),
            pltpu.VMEM((CS,), jnp.int32), pltpu.VMEM((CS,), jnp.int32),
            pltpu.VMEM((CS,), jnp.int32), pltpu.VMEM((CS,), jnp.int32),
            pltpu.VMEM((CS, H), jnp.float32), pltpu.VMEM((CS, H), jnp.float32),
            pltpu.SemaphoreType.DMA, pltpu.SemaphoreType.DMA,
        ],
        compiler_params=pltpu.CompilerParams(use_tc_tiling_on_sc=False),
        name="gnn_scatter_add",
    )


BN = 1568
BE = 5000
BP = 3136


def _encode_body(nf, new, neb, neg, nebb, wt, wb, h_o, a_o, b_o):
    y = jnp.dot(nf[...], new[...], preferred_element_type=jnp.float32)
    h = _gelu(_ln(y + neb[...], neg[...], nebb[...]))
    h_o[...] = h
    a_o[...] = jnp.dot(h, wt[...], preferred_element_type=jnp.float32)
    b_o[...] = jnp.dot(h, wb[...], preferred_element_type=jnp.float32)


def _encode_call(nf, new, neb, neg, nebb, wt, wb):
    grid = NP // BN
    vec = pl.BlockSpec((1, H), lambda i: (0, 0))
    return pl.pallas_call(
        _encode_body,
        grid=(grid,),
        in_specs=[
            pl.BlockSpec((BN, 8), lambda i: (i, 0)),
            pl.BlockSpec((8, H), lambda i: (0, 0)),
            vec, vec, vec,
            pl.BlockSpec((H, H), lambda i: (0, 0)),
            pl.BlockSpec((H, H), lambda i: (0, 0)),
        ],
        out_specs=[
            pl.BlockSpec((BN, H), lambda i: (i, 0)),
            pl.BlockSpec((BN, H), lambda i: (i, 0)),
            pl.BlockSpec((BN, H), lambda i: (i, 0)),
        ],
        out_shape=[jax.ShapeDtypeStruct((NP, H), jnp.float32)] * 3,
    )(nf, new, neb, neg, nebb, wt, wb)


def _lngelu_body(s_ref, mb, mg, mbb, m_o):
    y = s_ref[...] + mb[...]
    yl = y[:, :H]
    yr = y[:, H:]
    gl = _gelu(_ln(yl, mg[...][:, :H], mbb[...][:, :H]))
    gr = _gelu(_ln(yr, mg[...][:, H:], mbb[...][:, H:]))
    m_o[...] = jnp.concatenate([gl, gr], axis=1)


def _lngelu_call(s, mb, mg, mbb):
    s2 = s.reshape(E2 // 2, 2 * H)
    mb2 = jnp.concatenate([mb, mb], axis=1)
    mg2 = jnp.concatenate([mg, mg], axis=1)
    mbb2 = jnp.concatenate([mbb, mbb], axis=1)
    grid = (E2 // 2) // BE
    vec = pl.BlockSpec((1, 2 * H), lambda i: (0, 0))
    m2 = pl.pallas_call(
        _lngelu_body,
        grid=(grid,),
        in_specs=[pl.BlockSpec((BE, 2 * H), lambda i: (i, 0)), vec, vec, vec],
        out_specs=pl.BlockSpec((BE, 2 * H), lambda i: (i, 0)),
        out_shape=jax.ShapeDtypeStruct((E2 // 2, 2 * H), jnp.float32),
    )(s2, mb2, mg2, mbb2)
    return m2.reshape(E2, H)


def _update_body(h_ref, hn_ref, ng, nb, wt, wb, h_o, a_o, b_o):
    h = _ln(h_ref[...] + hn_ref[...], ng[...], nb[...])
    h_o[...] = h
    a_o[...] = jnp.dot(h, wt[...], preferred_element_type=jnp.float32)
    b_o[...] = jnp.dot(h, wb[...], preferred_element_type=jnp.float32)


def _update_call(h, hn, ng, nb, wt, wb):
    grid = NP // BN
    vec = pl.BlockSpec((1, H), lambda i: (0, 0))
    return pl.pallas_call(
        _update_body,
        grid=(grid,),
        in_specs=[
            pl.BlockSpec((BN, H), lambda i: (i, 0)),
            pl.BlockSpec((BN, H), lambda i: (i, 0)),
            vec, vec,
            pl.BlockSpec((H, H), lambda i: (0, 0)),
            pl.BlockSpec((H, H), lambda i: (0, 0)),
        ],
        out_specs=[
            pl.BlockSpec((BN, H), lambda i: (i, 0)),
            pl.BlockSpec((BN, H), lambda i: (i, 0)),
            pl.BlockSpec((BN, H), lambda i: (i, 0)),
        ],
        out_shape=[jax.ShapeDtypeStruct((NP, H), jnp.float32)] * 3,
    )(h, hn, ng, nb, wt, wb)


def _update_last_body(h_ref, hn_ref, ng, nb, h_o):
    h_o[...] = _ln(h_ref[...] + hn_ref[...], ng[...], nb[...])


def _update_last_call(h, hn, ng, nb):
    grid = NP // BN
    vec = pl.BlockSpec((1, H), lambda i: (0, 0))
    return pl.pallas_call(
        _update_last_body,
        grid=(grid,),
        in_specs=[
            pl.BlockSpec((BN, H), lambda i: (i, 0)),
            pl.BlockSpec((BN, H), lambda i: (i, 0)),
            vec, vec,
        ],
        out_specs=pl.BlockSpec((BN, H), lambda i: (i, 0)),
        out_shape=jax.ShapeDtypeStruct((NP, H), jnp.float32),
    )(h, hn, ng, nb)


def _pool_body(h_ref, nt_ref, gf, gew, geb, geg, gebb,
               pw1, pb1, pg1, pbb1, pw2, pb2, hw1, hb1, hw2, hb2,
               out_ref, acc_ref):
    i = pl.program_id(0)

    @pl.when(i == 0)
    def _():
        acc_ref[...] = jnp.zeros_like(acc_ref)

    h = h_ref[...]
    nt = nt_ref[...]
    qm = (nt == 0).astype(jnp.float32)
    cm = (nt == 1).astype(jnp.float32)
    acc_ref[0:1, 0:H] += jnp.sum(h * qm, axis=0, keepdims=True)
    acc_ref[1:2, 0:H] += jnp.sum(h * cm, axis=0, keepdims=True)
    acc_ref[2:3, 0:1] += jnp.sum(qm, keepdims=True)
    acc_ref[3:4, 0:1] += jnp.sum(cm, keepdims=True)

    @pl.when(i == NP // BP - 1)
    def _():
        qc = jnp.maximum(acc_ref[2, 0], 1.0)
        cc = jnp.maximum(acc_ref[3, 0], 1.0)
        q_pool = acc_ref[0:1, 0:H] / qc
        c_pool = acc_ref[1:2, 0:H] / cc
        gy = jnp.dot(gf[...], gew[...], preferred_element_type=jnp.float32)
        g = _gelu(_ln(gy + geb[...], geg[...], gebb[...]))
        combined = jnp.concatenate([q_pool, c_pool, g], axis=1)
        f1y = jnp.dot(combined, pw1[...], preferred_element_type=jnp.float32)
        f1 = _gelu(_ln(f1y + pb1[...], pg1[...], pbb1[...]))
        feats = _gelu(
            jnp.dot(f1, pw2[...], preferred_element_type=jnp.float32)
            + pb2[...])
        vals = []
        for hd in range(4):
            t = _gelu(
                jnp.dot(feats, hw1[hd], preferred_element_type=jnp.float32)
                + hb1[hd])
            vals.append(
                jnp.dot(t, hw2[hd], preferred_element_type=jnp.float32)
                + hb2[hd])
        acc = jax.nn.sigmoid(vals[0])
        en = jax.nn.softplus(vals[1])
        tr = jax.nn.sigmoid(vals[2])
        dp = jax.nn.softplus(vals[3])
        row = jnp.concatenate(
            [acc, en, tr, dp, jnp.zeros((1, 124), jnp.float32)], axis=1)
        out_ref[...] = row


def _pool_call(h, nt, gf, gew, geb, geg, gebb,
               pw1, pb1, pg1, pbb1, pw2, pb2, hw1, hb1, hw2, hb2):
    grid = NP // BP
    vec = pl.BlockSpec((1, H), lambda i: (0, 0))
    full = lambda shape: pl.BlockSpec(shape, lambda i: tuple(0 for _ in shape))
    return pl.pallas_call(
        _pool_body,
        grid=(grid,),
        in_specs=[
            pl.BlockSpec((BP, H), lambda i: (i, 0)),
            pl.BlockSpec((BP, 1), lambda i: (i, 0)),
            full((1, 8)), full((8, H)), vec, vec, vec,
            full((3 * H, 2 * H)), full((1, 2 * H)), full((1, 2 * H)),
            full((1, 2 * H)),
            full((2 * H, H)), full((1, H)),
            full((4, H, H // 2)), full((4, 1, H // 2)),
            full((4, H // 2, 1)), full((4, 1, 1)),
        ],
        out_specs=pl.BlockSpec((1, 128), lambda i: (0, 0)),
        out_shape=jax.ShapeDtypeStruct((1, 128), jnp.float32),
        scratch_shapes=[pltpu.VMEM((8, 128), jnp.float32)],
    )(h, nt, gf, gew, geb, geg, gebb,
      pw1, pb1, pg1, pbb1, pw2, pb2, hw1, hb1, hw2, hb2)


def kernel(node_features, edge_attr, global_features, edge_index, node_types,
           ne_w, ne_b, ne_g, ne_bb, ee_w, ee_b, ee_g, ee_bb,
           ge_w, ge_b, ge_g, ge_bb, msg_w, msg_b, msg_g, msg_bb,
           norm_g, norm_b, pw1, pb1, pg1, pbb1, pw2, pb2,
           hw1, hb1, hw2, hb2):
    del edge_attr, ee_w, ee_b, ee_g, ee_bb

    src = edge_index[0]
    dst = edge_index[1]
    nf_pad = jnp.zeros((NP, 8), jnp.float32).at[:N].set(node_features)
    nt_pad = jnp.full((NP, 1), 2, jnp.int32).at[:N, 0].set(node_types)

    r1 = lambda v: v.reshape(1, -1)
    wts = [msg_w[i, :H, :] for i in range(L)]
    wbs = [msg_w[i, H:, :] for i in range(L)]

    gather = _make_gather()
    scatter = _make_scatter()

    src0, src1 = src[:E2], src[E2:]
    dst0, dst1 = dst[:E2], dst[E2:]

    h, A, B = _encode_call(nf_pad, ne_w, r1(ne_b), r1(ne_g), r1(ne_bb),
                           wts[0], wbs[0])
    for i in range(L):
        S0 = gather(A, B, src0, dst0)
        S1 = gather(A, B, src1, dst1)
        M0 = _lngelu_call(S0, r1(msg_b[i]), r1(msg_g[i]), r1(msg_bb[i]))
        M1 = _lngelu_call(S1, r1(msg_b[i]), r1(msg_g[i]), r1(msg_bb[i]))
        hn = scatter(M0, M1, dst0, dst1)
        if i < L - 1:
            h, A, B = _update_call(h, hn, r1(norm_g[i]), r1(norm_b[i]),
                                   wts[i + 1], wbs[i + 1])
        else:
            h = _update_last_call(h, hn, r1(norm_g[i]), r1(norm_b[i]))

    pooled = _pool_call(h, nt_pad, global_features.reshape(1, 8),
                        ge_w, r1(ge_b), r1(ge_g), r1(ge_bb),
                        pw1, r1(pb1), r1(pg1), r1(pbb1), pw2, r1(pb2),
                        hw1, hb1.reshape(4, 1, H // 2), hw2,
                        hb2.reshape(4, 1, 1))
    return pooled[0, :4]


# speedup vs baseline: 1.3863x; 1.0003x over previous
import functools

import jax
import jax.numpy as jnp
from jax import lax
from jax.experimental import pallas as pl
from jax.experimental.pallas import tpu as pltpu
from jax.experimental.pallas import tpu_sc as plsc

N = 50000
E = 800000
H = 64
L = 4

NC = 2
NS = 16
NW = NC * NS

NP = 50176
NHALF = NP // 2

E2 = E // 2
CG = 128
NCH = E2 // CG
GFULL = NCH // NW
GEXTRA = NCH - GFULL * NW
CS = 128
SFULL = NCH // NS
SEXTRA = NCH - SFULL * NS

SROWS = NHALF + NS
OPS = NHALF // NS
OCH_FULL = OPS // CS
OREM = OPS - OCH_FULL * CS


def _ln(x, g, b):
    mu = jnp.mean(x, axis=-1, keepdims=True)
    xc = x - mu
    var = jnp.mean(xc * xc, axis=-1, keepdims=True)
    return xc / jnp.sqrt(var + 1e-5) * g + b


def _gelu(x):
    return 0.5 * x * (1.0 + lax.erf(x * 0.7071067811865476))


def _gather_body(a_hbm, b_hbm, src_hbm, dst_hbm, out_hbm,
                 idxa0, idxb0, idxa1, idxb1,
                 bufa0, bufb0, bufa1, bufb1,
                 sa0, sb0, sa1, sb1):
    c = lax.axis_index("c")
    s = lax.axis_index("s")
    wid = s * NC + c
    trip = jnp.where(wid < GEXTRA, GFULL + 1, GFULL)

    idxa = (idxa0, idxa1)
    idxb = (idxb0, idxb1)
    bufa = (bufa0, bufa1)
    bufb = (bufb0, bufb1)
    sa = (sa0, sa1)
    sb = (sb0, sb1)

    def fire(k, slot):
        off = (wid + k * NW) * CG
        pltpu.sync_copy(src_hbm.at[pl.ds(off, CG)], idxa[slot])
        pltpu.sync_copy(dst_hbm.at[pl.ds(off, CG)], idxb[slot])
        pltpu.async_copy(a_hbm.at[idxa[slot]], bufa[slot], sa[slot])
        pltpu.async_copy(b_hbm.at[idxb[slot]], bufb[slot], sb[slot])

    def drain_process(k, slot):
        pltpu.make_async_copy(a_hbm.at[idxa[slot]], bufa[slot], sa[slot]).wait()
        pltpu.make_async_copy(b_hbm.at[idxb[slot]], bufb[slot], sb[slot]).wait()
        A = bufa[slot]
        B = bufb[slot]

        def addrow(r, _):
            for j in range(H // 16):
                A[r, pl.ds(j * 16, 16)] = (
                    A[r, pl.ds(j * 16, 16)] + B[r, pl.ds(j * 16, 16)])
            return 0

        lax.fori_loop(0, CG, addrow, 0)
        pltpu.sync_copy(A, out_hbm.at[pl.ds((wid + k * NW) * CG, CG)])

    fire(0, 0)

    def step(k, _):
        @pl.when(k < trip)
        def _():
            @pl.when(k + 1 < trip)
            def _():
                @pl.when((k + 1) % 2 == 0)
                def _():
                    fire(k + 1, 0)

                @pl.when((k + 1) % 2 == 1)
                def _():
                    fire(k + 1, 1)

            @pl.when(k % 2 == 0)
            def _():
                drain_process(k, 0)

            @pl.when(k % 2 == 1)
            def _():
                drain_process(k, 1)

        return 0

    lax.fori_loop(0, GFULL + 1, step, 0)


def _make_gather():
    mesh = plsc.VectorSubcoreMesh(
        core_axis_name="c", subcore_axis_name="s",
        num_cores=NC, num_subcores=NS)
    return pl.kernel(
        _gather_body,
        out_type=jax.ShapeDtypeStruct((E2, H), jnp.float32),
        mesh=mesh,
        scratch_types=[
            pltpu.VMEM((CG,), jnp.int32), pltpu.VMEM((CG,), jnp.int32),
            pltpu.VMEM((CG,), jnp.int32), pltpu.VMEM((CG,), jnp.int32),
            pltpu.VMEM((CG, H), jnp.float32), pltpu.VMEM((CG, H), jnp.float32),
            pltpu.VMEM((CG, H), jnp.float32), pltpu.VMEM((CG, H), jnp.float32),
            pltpu.SemaphoreType.DMA, pltpu.SemaphoreType.DMA,
            pltpu.SemaphoreType.DMA, pltpu.SemaphoreType.DMA,
        ],
        compiler_params=pltpu.CompilerParams(use_tc_tiling_on_sc=False),
        name="gnn_edge_gather_add",
    )


def _scatter_body(m0_hbm, m1_hbm, dst0_hbm, dst1_hbm, out_hbm,
                  spmem, zbuf,
                  idxd0, idxd1, idxl0, idxl1, bufm0, bufm1,
                  sm0, sm1):
    c = lax.axis_index("c")
    s = lax.axis_index("s")
    lo = c * NHALF
    trash = NHALF + s
    trip = jnp.where(s < SEXTRA, SFULL + 1, SFULL)

    def zrow(r, _):
        for j in range(H // 16):
            zbuf[r, pl.ds(j * 16, 16)] = jnp.zeros((16,), jnp.float32)
        return 0

    lax.fori_loop(0, CS, zrow, 0)

    zb = s * OPS

    def zchunk(i, _):
        pltpu.sync_copy(zbuf, spmem.at[pl.ds(zb + i * CS, CS)])
        return 0

    lax.fori_loop(0, OCH_FULL, zchunk, 0)
    pltpu.sync_copy(zbuf.at[pl.ds(0, OREM)],
                    spmem.at[pl.ds(zb + OCH_FULL * CS, OREM)])
    plsc.subcore_barrier()

    idxd = (idxd0, idxd1)
    idxl = (idxl0, idxl1)
    bufm = (bufm0, bufm1)
    sm = (sm0, sm1)

    def run_half(m_hbm, dst_hbm):
        def fire(k, slot):
            off = (s + k * NS) * CS
            pltpu.sync_copy(dst_hbm.at[pl.ds(off, CS)], idxd[slot])
            pltpu.async_copy(m_hbm.at[pl.ds(off, CS)], bufm[slot], sm[slot])

        def drain_process(k, slot):
            def lslice(j, _):
                v = idxd[slot][pl.ds(j * 16, 16)]
                li = v - lo
                oob = (li < 0) | (li >= NHALF)
                idxl[slot][pl.ds(j * 16, 16)] = jnp.where(oob, trash, li)
                return 0

            lax.fori_loop(0, CS // 16, lslice, 0)
            pltpu.make_async_copy(
                m_hbm.at[pl.ds((s + k * NS) * CS, CS)],
                bufm[slot], sm[slot]).wait()
            pltpu.sync_copy(bufm[slot], spmem.at[idxl[slot]], add=True)

        fire(0, 0)

        def step(k, _):
            @pl.when(k < trip)
            def _():
                @pl.when(k + 1 < trip)
                def _():
                    @pl.when((k + 1) % 2 == 0)
                    def _():
                        fire(k + 1, 0)

                    @pl.when((k + 1) % 2 == 1)
                    def _():
                        fire(k + 1, 1)

                @pl.when(k % 2 == 0)
                def _():
                    drain_process(k, 0)

                @pl.when(k % 2 == 1)
                def _():
                    drain_process(k, 1)

            return 0

        lax.fori_loop(0, SFULL + 1, step, 0)

    @pl.when(c == 0)
    def _():
        run_half(m0_hbm, dst0_hbm)
        run_half(m1_hbm, dst1_hbm)

    @pl.when(c == 1)
    def _():
        run_half(m1_hbm, dst1_hbm)
        run_half(m0_hbm, dst0_hbm)

    plsc.subcore_barrier()

    ob = s * OPS

    def ochunk(i, _):
        pltpu.sync_copy(spmem.at[pl.ds(ob + i * CS, CS)],
                        out_hbm.at[pl.ds(lo + ob + i * CS, CS)])
        return 0

    lax.fori_loop(0, OCH_FULL, ochunk, 0)
    pltpu.sync_copy(spmem.at[pl.ds(ob + OCH_FULL * CS, OREM)],
                    out_hbm.at[pl.ds(lo + ob + OCH_FULL * CS, OREM)])


def _make_scatter():
    mesh = plsc.VectorSubcoreMesh(
        core_axis_name="c", subcore_axis_name="s",
        num_cores=NC, num_subcores=NS)
    return pl.kernel(
        _scatter_body,
        out_type=jax.ShapeDtypeStruct((NP, H), jnp.float32),
        mesh=mesh,
        scratch_types=[
            pltpu.VMEM_SHARED((SROWS, H), jnp.float32),
            pltpu.VMEM((CS, H), jnp.float32),
            pltpu.VMEM((CS,), jnp.int32), pltpu.VMEM((CS,), jnp.int32),
            pltpu.VMEM((CS,), jnp.int32), pltpu.VMEM((CS,), jnp.int32),
            pltpu.VMEM((CS, H), jnp.float32), pltpu.VMEM((CS, H), jnp.float32),
            pltpu.SemaphoreType.DMA, pltpu.SemaphoreType.DMA,
        ],
        compiler_params=pltpu.CompilerParams(use_tc_tiling_on_sc=False),
        name="gnn_scatter_add",
    )


BN = 1568
BE = 5000
BP = 3136


def _encode_body(nf, new, neb, neg, nebb, wt, wb, h_o, a_o, b_o):
    y = jnp.dot(nf[...], new[...], preferred_element_type=jnp.float32)
    h = _gelu(_ln(y + neb[...], neg[...], nebb[...]))
    h_o[...] = h
    a_o[...] = jnp.dot(h, wt[...], preferred_element_type=jnp.float32)
    b_o[...] = jnp.dot(h, wb[...], preferred_element_type=jnp.float32)


def _encode_call(nf, new, neb, neg, nebb, wt, wb):
    grid = NP // BN
    vec = pl.BlockSpec((1, H), lambda i: (0, 0))
    return pl.pallas_call(
        _encode_body,
        grid=(grid,),
        in_specs=[
            pl.BlockSpec((BN, 8), lambda i: (i, 0)),
            pl.BlockSpec((8, H), lambda i: (0, 0)),
            vec, vec, vec,
            pl.BlockSpec((H, H), lambda i: (0, 0)),
            pl.BlockSpec((H, H), lambda i: (0, 0)),
        ],
        out_specs=[
            pl.BlockSpec((BN, H), lambda i: (i, 0)),
            pl.BlockSpec((BN, H), lambda i: (i, 0)),
            pl.BlockSpec((BN, H), lambda i: (i, 0)),
        ],
        out_shape=[jax.ShapeDtypeStruct((NP, H), jnp.float32)] * 3,
    )(nf, new, neb, neg, nebb, wt, wb)


def _lngelu_body(s_ref, mb, mg, mbb, m_o):
    y = s_ref[...] + mb[...]
    yl = y[:, :H]
    yr = y[:, H:]
    gl = _gelu(_ln(yl, mg[...][:, :H], mbb[...][:, :H]))
    gr = _gelu(_ln(yr, mg[...][:, H:], mbb[...][:, H:]))
    m_o[...] = jnp.concatenate([gl, gr], axis=1)


def _lngelu_call(s, mb, mg, mbb):
    s2 = s.reshape(E2 // 2, 2 * H)
    mb2 = jnp.concatenate([mb, mb], axis=1)
    mg2 = jnp.concatenate([mg, mg], axis=1)
    mbb2 = jnp.concatenate([mbb, mbb], axis=1)
    grid = (E2 // 2) // BE
    vec = pl.BlockSpec((1, 2 * H), lambda i: (0, 0))
    m2 = pl.pallas_call(
        _lngelu_body,
        grid=(grid,),
        in_specs=[pl.BlockSpec((BE, 2 * H), lambda i: (i, 0)), vec, vec, vec],
        out_specs=pl.BlockSpec((BE, 2 * H), lambda i: (i, 0)),
        out_shape=jax.ShapeDtypeStruct((E2 // 2, 2 * H), jnp.float32),
    )(s2, mb2, mg2, mbb2)
    return m2.reshape(E2, H)


def _update_body(h_ref, hn_ref, ng, nb, wt, wb, h_o, a_o, b_o):
    h = _ln(h_ref[...] + hn_ref[...], ng[...], nb[...])
    h_o[...] = h
    a_o[...] = jnp.dot(h, wt[...], preferred_element_type=jnp.float32)
    b_o[...] = jnp.dot(h, wb[...], preferred_element_type=jnp.float32)


def _update_call(h, hn, ng, nb, wt, wb):
    grid = NP // BN
    vec = pl.BlockSpec((1, H), lambda i: (0, 0))
    return pl.pallas_call(
        _update_body,
        grid=(grid,),
        in_specs=[
            pl.BlockSpec((BN, H), lambda i: (i, 0)),
            pl.BlockSpec((BN, H), lambda i: (i, 0)),
            vec, vec,
            pl.BlockSpec((H, H), lambda i: (0, 0)),
            pl.BlockSpec((H, H), lambda i: (0, 0)),
        ],
        out_specs=[
            pl.BlockSpec((BN, H), lambda i: (i, 0)),
            pl.BlockSpec((BN, H), lambda i: (i, 0)),
            pl.BlockSpec((BN, H), lambda i: (i, 0)),
        ],
        out_shape=[jax.ShapeDtypeStruct((NP, H), jnp.float32)] * 3,
    )(h, hn, ng, nb, wt, wb)


def _update_last_body(h_ref, hn_ref, ng, nb, h_o):
    h_o[...] = _ln(h_ref[...] + hn_ref[...], ng[...], nb[...])


def _update_last_call(h, hn, ng, nb):
    grid = NP // BN
    vec = pl.BlockSpec((1, H), lambda i: (0, 0))
    return pl.pallas_call(
        _update_last_body,
        grid=(grid,),
        in_specs=[
            pl.BlockSpec((BN, H), lambda i: (i, 0)),
            pl.BlockSpec((BN, H), lambda i: (i, 0)),
            vec, vec,
        ],
        out_specs=pl.BlockSpec((BN, H), lambda i: (i, 0)),
        out_shape=jax.ShapeDtypeStruct((NP, H), jnp.float32),
    )(h, hn, ng, nb)


def _pool_body(h_ref, nt_ref, gf, gew, geb, geg, gebb,
               pw1, pb1, pg1, pbb1, pw2, pb2, hw1, hb1, hw2, hb2,
               out_ref, acc_ref):
    i = pl.program_id(0)

    @pl.when(i == 0)
    def _():
        acc_ref[...] = jnp.zeros_like(acc_ref)

    h = h_ref[...]
    nt = nt_ref[...]
    qm = (nt == 0).astype(jnp.float32)
    cm = (nt == 1).astype(jnp.float32)
    acc_ref[0:1, 0:H] += jnp.sum(h * qm, axis=0, keepdims=True)
    acc_ref[1:2, 0:H] += jnp.sum(h * cm, axis=0, keepdims=True)
    acc_ref[2:3, 0:1] += jnp.sum(qm, keepdims=True)
    acc_ref[3:4, 0:1] += jnp.sum(cm, keepdims=True)

    @pl.when(i == NP // BP - 1)
    def _():
        qc = jnp.maximum(acc_ref[2, 0], 1.0)
        cc = jnp.maximum(acc_ref[3, 0], 1.0)
        q_pool = acc_ref[0:1, 0:H] / qc
        c_pool = acc_ref[1:2, 0:H] / cc
        gy = jnp.dot(gf[...], gew[...], preferred_element_type=jnp.float32)
        g = _gelu(_ln(gy + geb[...], geg[...], gebb[...]))
        combined = jnp.concatenate([q_pool, c_pool, g], axis=1)
        f1y = jnp.dot(combined, pw1[...], preferred_element_type=jnp.float32)
        f1 = _gelu(_ln(f1y + pb1[...], pg1[...], pbb1[...]))
        feats = _gelu(
            jnp.dot(f1, pw2[...], preferred_element_type=jnp.float32)
            + pb2[...])
        vals = []
        for hd in range(4):
            t = _gelu(
                jnp.dot(feats, hw1[hd], preferred_element_type=jnp.float32)
                + hb1[hd])
            vals.append(
                jnp.dot(t, hw2[hd], preferred_element_type=jnp.float32)
                + hb2[hd])
        acc = jax.nn.sigmoid(vals[0])
        en = jax.nn.softplus(vals[1])
        tr = jax.nn.sigmoid(vals[2])
        dp = jax.nn.softplus(vals[3])
        row = jnp.concatenate(
            [acc, en, tr, dp, jnp.zeros((1, 124), jnp.float32)], axis=1)
        out_ref[...] = row


def _pool_call(h, nt, gf, gew, geb, geg, gebb,
               pw1, pb1, pg1, pbb1, pw2, pb2, hw1, hb1, hw2, hb2):
    grid = NP // BP
    vec = pl.BlockSpec((1, H), lambda i: (0, 0))
    full = lambda shape: pl.BlockSpec(shape, lambda i: tuple(0 for _ in shape))
    return pl.pallas_call(
        _pool_body,
        grid=(grid,),
        in_specs=[
            pl.BlockSpec((BP, H), lambda i: (i, 0)),
            pl.BlockSpec((BP, 1), lambda i: (i, 0)),
            full((1, 8)), full((8, H)), vec, vec, vec,
            full((3 * H, 2 * H)), full((1, 2 * H)), full((1, 2 * H)),
            full((1, 2 * H)),
            full((2 * H, H)), full((1, H)),
            full((4, H, H // 2)), full((4, 1, H // 2)),
            full((4, H // 2, 1)), full((4, 1, 1)),
        ],
        out_specs=pl.BlockSpec((1, 128), lambda i: (0, 0)),
        out_shape=jax.ShapeDtypeStruct((1, 128), jnp.float32),
        scratch_shapes=[pltpu.VMEM((8, 128), jnp.float32)],
    )(h, nt, gf, gew, geb, geg, gebb,
      pw1, pb1, pg1, pbb1, pw2, pb2, hw1, hb1, hw2, hb2)


def kernel(node_features, edge_attr, global_features, edge_index, node_types,
           ne_w, ne_b, ne_g, ne_bb, ee_w, ee_b, ee_g, ee_bb,
           ge_w, ge_b, ge_g, ge_bb, msg_w, msg_b, msg_g, msg_bb,
           norm_g, norm_b, pw1, pb1, pg1, pbb1, pw2, pb2,
           hw1, hb1, hw2, hb2):
    del edge_attr, ee_w, ee_b, ee_g, ee_bb

    src = edge_index[0]
    dst = edge_index[1]
    nf_pad = jnp.zeros((NP, 8), jnp.float32).at[:N].set(node_features)
    nt_pad = jnp.full((NP, 1), 2, jnp.int32).at[:N, 0].set(node_types)

    r1 = lambda v: v.reshape(1, -1)
    wts = [msg_w[i, :H, :] for i in range(L)]
    wbs = [msg_w[i, H:, :] for i in range(L)]

    gather = _make_gather()
    scatter = _make_scatter()

    src0, src1 = src[:E2], src[E2:]
    dst0, dst1 = dst[:E2], dst[E2:]

    h, A, B = _encode_call(nf_pad, ne_w, r1(ne_b), r1(ne_g), r1(ne_bb),
                           wts[0], wbs[0])
    for i in range(L):
        S0 = gather(A, B, src0, dst0)
        S1 = gather(A, B, src1, dst1)
        M0 = _lngelu_call(S0, r1(msg_b[i]), r1(msg_g[i]), r1(msg_bb[i]))
        M1 = _lngelu_call(S1, r1(msg_b[i]), r1(msg_g[i]), r1(msg_bb[i]))
        hn = scatter(M0, M1, dst0, dst1)
        if i < L - 1:
            h, A, B = _update_call(h, hn, r1(norm_g[i]), r1(norm_b[i]),
                                   wts[i + 1], wbs[i + 1])
        else:
            h = _update_last_call(h, hn, r1(norm_g[i]), r1(norm_b[i]))

    pooled = _pool_call(h, nt_pad, global_features.reshape(1, 8),
                        ge_w, r1(ge_b), r1(ge_g), r1(ge_bb),
                        pw1, r1(pb1), r1(pg1), r1(pbb1), pw2, r1(pb2),
                        hw1, hb1.reshape(4, 1, H // 2), hw2,
                        hb2.reshape(4, 1, 1))
    return pooled[0, :4]
